# baseline (speedup 1.0000x reference)
import functools

import jax
import jax.numpy as jnp
from jax.experimental import pallas as pl
from jax.experimental.pallas import tpu as pltpu


def _dma_body(batch, w_hbm, o_hbm, sems):
    copies = [
        pltpu.make_async_copy(w_hbm, o_hbm.at[b], sems.at[b])
        for b in range(batch)
    ]
    for c in copies:
        c.start()
    for c in copies:
        c.wait()


def kernel(tokens, W_pos):
    batch, seq = tokens.shape
    d = W_pos.shape[-1]
    pos = W_pos[:seq]
    return pl.pallas_call(
        functools.partial(_dma_body, batch),
        in_specs=[pl.BlockSpec(memory_space=pl.ANY)],
        out_specs=pl.BlockSpec(memory_space=pl.ANY),
        out_shape=jax.ShapeDtypeStruct((batch, seq, d), W_pos.dtype),
        scratch_shapes=[
            pltpu.SemaphoreType.DMA((batch,)),
        ],
    )(pos)


# speedup vs baseline: 54.3534x; 54.3534x over previous
import functools

import jax
import jax.numpy as jnp
from jax import lax
from jax.experimental import pallas as pl
from jax.experimental.pallas import tpu as pltpu
from jax.experimental.pallas import tpu_sc as plsc

_CHUNK = 32
_NBUF = 3


def _sc_body(batch, rows_per_w, chunk, nbuf, nc, w_hbm, o_hbm, buf, rsems, wsems):
    wid = lax.axis_index("s") * nc + lax.axis_index("c")
    base = wid * rows_per_w
    n = rows_per_w // chunk

    def read(i):
        return pltpu.make_async_copy(
            w_hbm.at[pl.ds(base + i * chunk, chunk)],
            buf.at[i % nbuf], rsems.at[i % nbuf])

    def write(i, b):
        return pltpu.make_async_copy(
            buf.at[i % nbuf],
            o_hbm.at[b, pl.ds(base + i * chunk, chunk)], wsems.at[i % nbuf])

    for i in range(min(nbuf - 1, n)):
        read(i).start()
    for i in range(n):
        read(i).wait()
        for b in range(batch):
            write(i, b).start()
        j = i + nbuf - 1
        if j < n:
            if j - nbuf >= 0:
                for b in range(batch):
                    write(j - nbuf, b).wait()
            read(j).start()
    for i in range(max(0, n - nbuf), n):
        for b in range(batch):
            write(i, b).wait()


def kernel(tokens, W_pos):
    batch, seq = tokens.shape
    d = W_pos.shape[-1]
    pos = W_pos[:seq]
    info = plsc.get_sparse_core_info()
    nc, ns = info.num_cores, info.num_subcores
    nw = nc * ns
    rows_per_w = seq // nw
    chunk = min(_CHUNK, rows_per_w)
    mesh = plsc.VectorSubcoreMesh(core_axis_name="c", subcore_axis_name="s")
    k = functools.partial(
        pl.kernel,
        mesh=mesh,
        out_type=jax.ShapeDtypeStruct((batch, seq, d), W_pos.dtype),
        scratch_types=[
            pltpu.VMEM((_NBUF, chunk, d), W_pos.dtype),
            pltpu.SemaphoreType.DMA((_NBUF,)),
            pltpu.SemaphoreType.DMA((_NBUF,)),
        ],
    )(functools.partial(_sc_body, batch, rows_per_w, chunk, _NBUF, nc))
    return k(pos)
